# Initial kernel scaffold; baseline (speedup 1.0000x reference)
import jax
import jax.numpy as jnp
from jax.experimental import pallas as pl


def kernel(inputs, table):
    raise NotImplementedError("write your pallas kernel here")


# speedup vs baseline: 4.0854x; 4.0854x over previous
import functools

import jax
import jax.numpy as jnp
from jax import lax
from jax.experimental import pallas as pl
from jax.experimental.pallas import tpu as pltpu
from jax.experimental.pallas import tpu_sc as plsc

VOCAB = 100000
EMBED_DIM = 64
NUM_WORKERS = 32
CHUNK = 128


def _make_lookup(n_chunks):
    mesh = plsc.VectorSubcoreMesh(core_axis_name="c", subcore_axis_name="s")

    @functools.partial(
        pl.kernel,
        mesh=mesh,
        out_type=jax.ShapeDtypeStruct(
            (NUM_WORKERS, n_chunks, CHUNK, EMBED_DIM), jnp.float32
        ),
        scratch_types=[
            pltpu.VMEM((n_chunks, CHUNK), jnp.int32),
            pltpu.VMEM((CHUNK, EMBED_DIM), jnp.float32),
            pltpu.SemaphoreType.DMA,
        ],
        compiler_params=pltpu.CompilerParams(use_tc_tiling_on_sc=False),
    )
    def lookup(table_hbm, idx_hbm, out_hbm, idx_v, rows_v, sem):
        wid = lax.axis_index("s") * 2 + lax.axis_index("c")
        pltpu.sync_copy(idx_hbm.at[wid], idx_v)

        def body(j, _):
            pltpu.async_copy(table_hbm.at[idx_v.at[j]], rows_v, sem).wait()
            pltpu.sync_copy(rows_v, out_hbm.at[wid, j])
            return _

        lax.fori_loop(0, n_chunks, body, None)

    return lookup


def kernel(inputs, table):
    batch, seq = inputs.shape
    total = batch * seq
    n_chunks = total // (NUM_WORKERS * CHUNK)
    idx = inputs.reshape(NUM_WORKERS, n_chunks, CHUNK).astype(jnp.int32)
    out = _make_lookup(n_chunks)(table, idx)
    return out.reshape(batch, seq, EMBED_DIM)


# speedup vs baseline: 4.6412x; 1.1361x over previous
import functools

import jax
import jax.numpy as jnp
from jax import lax
from jax.experimental import pallas as pl
from jax.experimental.pallas import tpu as pltpu
from jax.experimental.pallas import tpu_sc as plsc

VOCAB = 100000
EMBED_DIM = 64
NUM_WORKERS = 32
CHUNK = 128
NBUF = 5


def _make_lookup(n_chunks):
    assert n_chunks % NBUF == 0
    n_groups = n_chunks // NBUF
    mesh = plsc.VectorSubcoreMesh(core_axis_name="c", subcore_axis_name="s")

    @functools.partial(
        pl.kernel,
        mesh=mesh,
        out_type=jax.ShapeDtypeStruct(
            (NUM_WORKERS, n_chunks, CHUNK, EMBED_DIM), jnp.float32
        ),
        scratch_types=[
            pltpu.VMEM((n_chunks, CHUNK), jnp.int32),
            pltpu.VMEM((NBUF, CHUNK, EMBED_DIM), jnp.float32),
            pltpu.SemaphoreType.DMA((NBUF,)),
            pltpu.SemaphoreType.DMA((NBUF,)),
        ],
        compiler_params=pltpu.CompilerParams(use_tc_tiling_on_sc=False),
    )
    def lookup(table_hbm, idx_hbm, out_hbm, idx_v, rows_v, gsem, ssem):
        wid = lax.axis_index("s") * 2 + lax.axis_index("c")
        pltpu.sync_copy(idx_hbm.at[wid], idx_v)

        def gather_start(j, b):
            pltpu.async_copy(table_hbm.at[idx_v.at[j]], rows_v.at[b], gsem.at[b])

        def gather_wait(j, b):
            pltpu.make_async_copy(
                table_hbm.at[idx_v.at[j]], rows_v.at[b], gsem.at[b]
            ).wait()

        def store_start(j, b):
            pltpu.async_copy(rows_v.at[b], out_hbm.at[wid, j], ssem.at[b])

        def store_wait(j, b):
            pltpu.make_async_copy(
                rows_v.at[b], out_hbm.at[wid, j], ssem.at[b]
            ).wait()

        for b in range(NBUF):
            gather_start(b, b)

        def group(g, carry):
            prev = (g - 1) * NBUF
            cur = g * NBUF
            for b in range(NBUF):
                gather_wait(prev + b, b)
                store_start(prev + b, b)
            for b in range(NBUF):
                store_wait(prev + b, b)
                gather_start(cur + b, b)
            return carry

        lax.fori_loop(1, n_groups, group, None)

        last = (n_groups - 1) * NBUF
        for b in range(NBUF):
            gather_wait(last + b, b)
            pltpu.sync_copy(rows_v.at[b], out_hbm.at[wid, last + b])

    return lookup


def kernel(inputs, table):
    batch, seq = inputs.shape
    total = batch * seq
    n_chunks = total // (NUM_WORKERS * CHUNK)
    idx = inputs.reshape(NUM_WORKERS, n_chunks, CHUNK).astype(jnp.int32)
    out = _make_lookup(n_chunks)(table, idx)
    return out.reshape(batch, seq, EMBED_DIM)
